# Initial kernel scaffold; baseline (speedup 1.0000x reference)
import jax
import jax.numpy as jnp
from jax.experimental import pallas as pl


def kernel(x, edge_index, W, b):
    raise NotImplementedError("write your pallas kernel here")


# speedup vs baseline: 8.1188x; 8.1188x over previous
import functools

import jax
import jax.numpy as jnp
from jax import lax
from jax.experimental import pallas as pl
from jax.experimental.pallas import tpu as pltpu
from jax.experimental.pallas import tpu_sc as plsc

NC = 2
NS = 16
NW = NC * NS


def _sc_segment_sum(x, src3, dst3, zer):
    n, d = x.shape
    npad = zer.shape[0] * NS
    _, kch, chunk = src3.shape
    rps = npad // NS
    mesh = plsc.VectorSubcoreMesh(core_axis_name="c", subcore_axis_name="s")

    @functools.partial(
        pl.kernel,
        out_type=jax.ShapeDtypeStruct((NC, npad, d), jnp.float32),
        mesh=mesh,
        scratch_types=[
            pltpu.VMEM((kch, chunk), jnp.int32),
            pltpu.VMEM((kch, chunk), jnp.int32),
            pltpu.VMEM((chunk, d), jnp.float32),
            pltpu.VMEM_SHARED((npad, d), jnp.float32),
        ],
    )
    def seg_sum(x_hbm, src_hbm, dst_hbm, zer_hbm, out_hbm, src_v, dst_v,
                rows_v, acc_sh):
        c = lax.axis_index("c")
        s = lax.axis_index("s")
        wid = c * NS + s

        pltpu.sync_copy(zer_hbm, acc_sh.at[pl.ds(s * rps, rps)])
        pltpu.sync_copy(src_hbm.at[wid], src_v)
        pltpu.sync_copy(dst_hbm.at[wid], dst_v)
        plsc.subcore_barrier()

        @pl.loop(0, kch)
        def _(k):
            pltpu.sync_copy(x_hbm.at[src_v.at[k]], rows_v)
            pltpu.sync_copy(rows_v, acc_sh.at[dst_v.at[k]], add=True)

        plsc.subcore_barrier()
        pltpu.sync_copy(acc_sh.at[pl.ds(s * rps, rps)],
                        out_hbm.at[c, pl.ds(s * rps, rps)])

    return seg_sum(x, src3, dst3, zer)


def _tc_linear(x, parts, w1t, w2t, b2):
    n, d = x.shape
    out = w1t.shape[1]
    bm = 1000

    def lin(x_ref, p_ref, w1_ref, w2_ref, b_ref, o_ref):
        sup = p_ref[0] + p_ref[1]
        acc = jnp.dot(x_ref[...], w1_ref[...], preferred_element_type=jnp.float32)
        acc += jnp.dot(sup, w2_ref[...], preferred_element_type=jnp.float32)
        o_ref[...] = jnp.maximum(acc + b_ref[...], 0.0)

    return pl.pallas_call(
        lin,
        grid=(n // bm,),
        in_specs=[
            pl.BlockSpec((bm, d), lambda i: (i, 0)),
            pl.BlockSpec((NC, bm, d), lambda i: (0, i, 0)),
            pl.BlockSpec((d, out), lambda i: (0, 0)),
            pl.BlockSpec((d, out), lambda i: (0, 0)),
            pl.BlockSpec((1, out), lambda i: (0, 0)),
        ],
        out_specs=pl.BlockSpec((bm, out), lambda i: (i, 0)),
        out_shape=jax.ShapeDtypeStruct((n, out), jnp.float32),
    )(x, parts, w1t, w2t, b2)


def kernel(x, edge_index, W, b):
    n, d = x.shape
    e = edge_index.shape[1]
    epw = e // NW
    chunk = 100
    kch = epw // chunk

    ei = edge_index.astype(jnp.int32)
    src3 = ei[0].reshape(NW, kch, chunk)
    dst3 = ei[1].reshape(NW, kch, chunk)
    rps = -(-n // (NS * 8)) * 8
    zer = jnp.zeros((rps, d), jnp.float32)

    parts = _sc_segment_sum(x, src3, dst3, zer)

    wt = W.T
    w1t = wt[:d]
    w2t = wt[d:]
    b2 = b.reshape(1, -1)
    return _tc_linear(x, parts, w1t, w2t, b2)


# speedup vs baseline: 10.6742x; 1.3148x over previous
import functools

import jax
import jax.numpy as jnp
from jax import lax
from jax.experimental import pallas as pl
from jax.experimental.pallas import tpu as pltpu
from jax.experimental.pallas import tpu_sc as plsc

NC = 2
NS = 16
NW = NC * NS


def _sc_segment_sum(x, src3, dst3, zer):
    n, d = x.shape
    npad = zer.shape[0] * NS
    _, kblk, blk, chunk = dst3.shape
    rps = npad // NS
    mesh = plsc.VectorSubcoreMesh(core_axis_name="c", subcore_axis_name="s")

    @functools.partial(
        pl.kernel,
        out_type=jax.ShapeDtypeStruct((NC, npad, d), jnp.float32),
        mesh=mesh,
        scratch_types=[
            pltpu.VMEM((blk, chunk), jnp.int32),
            pltpu.VMEM((blk, chunk), jnp.int32),
            pltpu.VMEM((chunk, d), jnp.float32),
            pltpu.VMEM((chunk, d), jnp.float32),
            pltpu.VMEM_SHARED((npad, d), jnp.float32),
            pltpu.SemaphoreType.DMA,
            pltpu.SemaphoreType.DMA,
        ],
    )
    def seg_sum(x_hbm, src_hbm, dst_hbm, zer_hbm, out_hbm, src_v, dst_v,
                rows0_v, rows1_v, acc_sh, sem0, sem1):
        c = lax.axis_index("c")
        s = lax.axis_index("s")
        wid = c * NS + s
        bufs = ((rows0_v, sem0), (rows1_v, sem1))
        pltpu.sync_copy(zer_hbm, acc_sh.at[pl.ds(s * rps, rps)])
        plsc.subcore_barrier()

        @pl.loop(0, kblk)
        def _(j):
            pltpu.sync_copy(src_hbm.at[wid, j], src_v)
            pltpu.sync_copy(dst_hbm.at[wid, j], dst_v)
            gathers = [None] * blk
            gathers[0] = pltpu.make_async_copy(
                x_hbm.at[src_v.at[0]], rows0_v, sem0)
            gathers[0].start()
            for r in range(blk):
                buf, _ = bufs[r % 2]
                if r + 1 < blk:
                    nbuf, nsem = bufs[(r + 1) % 2]
                    gathers[r + 1] = pltpu.make_async_copy(
                        x_hbm.at[src_v.at[r + 1]], nbuf, nsem)
                    gathers[r + 1].start()
                gathers[r].wait()
                pltpu.sync_copy(buf, acc_sh.at[dst_v.at[r]], add=True)

        plsc.subcore_barrier()
        pltpu.sync_copy(acc_sh.at[pl.ds(s * rps, rps)],
                        out_hbm.at[c, pl.ds(s * rps, rps)])

    return seg_sum(x, src3, dst3, zer)


def _tc_linear(x, parts, w1t, w2t, b2):
    n, d = x.shape
    out = w1t.shape[1]
    bm = 1000

    def lin(x_ref, p_ref, w1_ref, w2_ref, b_ref, o_ref):
        sup = p_ref[0] + p_ref[1]
        acc = jnp.dot(x_ref[...], w1_ref[...], preferred_element_type=jnp.float32)
        acc += jnp.dot(sup, w2_ref[...], preferred_element_type=jnp.float32)
        o_ref[...] = jnp.maximum(acc + b_ref[...], 0.0)

    return pl.pallas_call(
        lin,
        grid=(n // bm,),
        in_specs=[
            pl.BlockSpec((bm, d), lambda i: (i, 0)),
            pl.BlockSpec((NC, bm, d), lambda i: (0, i, 0)),
            pl.BlockSpec((d, out), lambda i: (0, 0)),
            pl.BlockSpec((d, out), lambda i: (0, 0)),
            pl.BlockSpec((1, out), lambda i: (0, 0)),
        ],
        out_specs=pl.BlockSpec((bm, out), lambda i: (i, 0)),
        out_shape=jax.ShapeDtypeStruct((n, out), jnp.float32),
    )(x, parts, w1t, w2t, b2)


def kernel(x, edge_index, W, b):
    n, d = x.shape
    e = edge_index.shape[1]
    epw = e // NW
    chunk = 100
    blk = 10
    kblk = epw // (blk * chunk)

    ei = edge_index.astype(jnp.int32)
    src3 = ei[0].reshape(NW, kblk, blk, chunk)
    dst3 = ei[1].reshape(NW, kblk, blk, chunk)
    rps = -(-n // (NS * 8)) * 8
    zer = jnp.zeros((rps, d), jnp.float32)

    parts = _sc_segment_sum(x, src3, dst3, zer)

    wt = W.T
    w1t = wt[:d]
    w2t = wt[d:]
    b2 = b.reshape(1, -1)
    return _tc_linear(x, parts, w1t, w2t, b2)


# speedup vs baseline: 12.3676x; 1.1586x over previous
import functools

import jax
import jax.numpy as jnp
from jax import lax
from jax.experimental import pallas as pl
from jax.experimental.pallas import tpu as pltpu
from jax.experimental.pallas import tpu_sc as plsc

NC = 2
NS = 16
NW = NC * NS


def _sc_segment_sum(x, src3, dst3, zer):
    n, d = x.shape
    npad = zer.shape[0] * NS
    _, kblk, blk, chunk = dst3.shape
    rps = npad // NS
    mesh = plsc.VectorSubcoreMesh(core_axis_name="c", subcore_axis_name="s")

    @functools.partial(
        pl.kernel,
        out_type=jax.ShapeDtypeStruct((NC, npad, d), jnp.float32),
        mesh=mesh,
        scratch_types=[
            pltpu.VMEM((blk, chunk), jnp.int32),
            pltpu.VMEM((blk, chunk), jnp.int32),
            pltpu.VMEM((chunk, d), jnp.float32),
            pltpu.VMEM((chunk, d), jnp.float32),
            pltpu.VMEM((chunk, d), jnp.float32),
            pltpu.VMEM_SHARED((npad, d), jnp.float32),
            pltpu.SemaphoreType.DMA,
            pltpu.SemaphoreType.DMA,
            pltpu.SemaphoreType.DMA,
        ],
    )
    def seg_sum(x_hbm, src_hbm, dst_hbm, zer_hbm, out_hbm, src_v, dst_v,
                rows0_v, rows1_v, rows2_v, acc_sh, sem0, sem1, sem2):
        c = lax.axis_index("c")
        s = lax.axis_index("s")
        wid = c * NS + s
        bufs = ((rows0_v, sem0), (rows1_v, sem1), (rows2_v, sem2))
        nbufs = len(bufs)
        pltpu.sync_copy(zer_hbm, acc_sh.at[pl.ds(s * rps, rps)])
        plsc.subcore_barrier()

        @pl.loop(0, kblk)
        def _(j):
            pltpu.sync_copy(src_hbm.at[wid, j], src_v)
            pltpu.sync_copy(dst_hbm.at[wid, j], dst_v)
            gathers = [None] * blk
            for r in range(nbufs - 1):
                buf, sem = bufs[r]
                gathers[r] = pltpu.make_async_copy(
                    x_hbm.at[src_v.at[r]], buf, sem)
                gathers[r].start()
            for r in range(blk):
                buf, _ = bufs[r % nbufs]
                if r + nbufs - 1 < blk:
                    nbuf, nsem = bufs[(r + nbufs - 1) % nbufs]
                    gathers[r + nbufs - 1] = pltpu.make_async_copy(
                        x_hbm.at[src_v.at[r + nbufs - 1]], nbuf, nsem)
                    gathers[r + nbufs - 1].start()
                gathers[r].wait()
                pltpu.sync_copy(buf, acc_sh.at[dst_v.at[r]], add=True)

        plsc.subcore_barrier()
        pltpu.sync_copy(acc_sh.at[pl.ds(s * rps, rps)],
                        out_hbm.at[c, pl.ds(s * rps, rps)])

    return seg_sum(x, src3, dst3, zer)


def _tc_linear(x, parts, w1t, w2t, b2):
    n, d = x.shape
    out = w1t.shape[1]
    bm = 1000

    def lin(x_ref, p_ref, w1_ref, w2_ref, b_ref, o_ref):
        sup = p_ref[0] + p_ref[1]
        acc = jnp.dot(x_ref[...], w1_ref[...], preferred_element_type=jnp.float32)
        acc += jnp.dot(sup, w2_ref[...], preferred_element_type=jnp.float32)
        o_ref[...] = jnp.maximum(acc + b_ref[...], 0.0)

    return pl.pallas_call(
        lin,
        grid=(n // bm,),
        in_specs=[
            pl.BlockSpec((bm, d), lambda i: (i, 0)),
            pl.BlockSpec((NC, bm, d), lambda i: (0, i, 0)),
            pl.BlockSpec((d, out), lambda i: (0, 0)),
            pl.BlockSpec((d, out), lambda i: (0, 0)),
            pl.BlockSpec((1, out), lambda i: (0, 0)),
        ],
        out_specs=pl.BlockSpec((bm, out), lambda i: (i, 0)),
        out_shape=jax.ShapeDtypeStruct((n, out), jnp.float32),
    )(x, parts, w1t, w2t, b2)


def kernel(x, edge_index, W, b):
    n, d = x.shape
    e = edge_index.shape[1]
    epw = e // NW
    chunk = 100
    blk = 20
    kblk = epw // (blk * chunk)

    ei = edge_index.astype(jnp.int32)
    src3 = ei[0].reshape(NW, kblk, blk, chunk)
    dst3 = ei[1].reshape(NW, kblk, blk, chunk)
    rps = -(-n // (NS * 8)) * 8
    zer = jnp.zeros((rps, d), jnp.float32)

    parts = _sc_segment_sum(x, src3, dst3, zer)

    wt = W.T
    w1t = wt[:d]
    w2t = wt[d:]
    b2 = b.reshape(1, -1)
    return _tc_linear(x, parts, w1t, w2t, b2)
